# initial kernel scaffold (unmeasured)
import jax
import jax.numpy as jnp
from jax import lax
from jax.experimental import pallas as pl
from jax.experimental.pallas import tpu as pltpu


def kernel(
    x,
):
    def body(*refs):
        pass

    out_shape = jax.ShapeDtypeStruct(..., jnp.float32)
    return pl.pallas_call(body, out_shape=out_shape)(...)


# baseline (device time: 17803 ns/iter reference)
import jax
import jax.numpy as jnp
from jax import lax
from jax.experimental import pallas as pl
from jax.experimental.pallas import tpu as pltpu


def kernel(x):
    m, n = x.shape

    def body(x_ref, out_ref, send_buf, recv_buf, send_sem, recv_sem):
        my_x = lax.axis_index("x")
        my_y = lax.axis_index("y")
        my_z = lax.axis_index("z")
        other_x = 1 - my_x
        nbr = (other_x, my_y, my_z)

        barrier_sem = pltpu.get_barrier_semaphore()
        pl.semaphore_signal(
            barrier_sem, inc=1, device_id=nbr,
            device_id_type=pl.DeviceIdType.MESH,
        )
        pl.semaphore_wait(barrier_sem, 1)

        send_buf[...] = x_ref[...].astype(jnp.bfloat16)
        rdma = pltpu.make_async_remote_copy(
            src_ref=send_buf,
            dst_ref=recv_buf,
            send_sem=send_sem,
            recv_sem=recv_sem,
            device_id=nbr,
            device_id_type=pl.DeviceIdType.MESH,
        )
        rdma.start()
        out_ref[pl.ds(my_x * m, m), :] = send_buf[...]
        rdma.wait()
        out_ref[pl.ds(other_x * m, m), :] = recv_buf[...]

    return pl.pallas_call(
        body,
        out_shape=jax.ShapeDtypeStruct((2 * m, n), jnp.bfloat16),
        in_specs=[pl.BlockSpec(memory_space=pltpu.VMEM)],
        out_specs=pl.BlockSpec(memory_space=pltpu.VMEM),
        scratch_shapes=[
            pltpu.VMEM((m, n), jnp.bfloat16),
            pltpu.VMEM((m, n), jnp.bfloat16),
            pltpu.SemaphoreType.DMA,
            pltpu.SemaphoreType.DMA,
        ],
        compiler_params=pltpu.CompilerParams(collective_id=0),
    )(x)


# device time: 15418 ns/iter; 1.1547x vs baseline; 1.1547x over previous
import jax
import jax.numpy as jnp
from jax import lax
from jax.experimental import pallas as pl
from jax.experimental.pallas import tpu as pltpu

CH = 8


def kernel(x):
    m, n = x.shape
    half = m // 2
    r = half // CH

    def body(x_ref, out_ref, s1_send, s1_recv, s2_send, s2_recv):
        my_x = lax.axis_index("x")
        my_y = lax.axis_index("y")
        my_z = lax.axis_index("z")
        xn = (1 - my_x, my_y, my_z)
        yn = (my_x, 1 - my_y, my_z)

        barrier_sem = pltpu.get_barrier_semaphore()
        for nbr in (xn, yn):
            pl.semaphore_signal(
                barrier_sem, inc=1, device_id=nbr,
                device_id_type=pl.DeviceIdType.MESH,
            )
        own = my_x * m
        opp = (1 - my_x) * m
        out_ref[pl.ds(own, m), :] = x_ref[...].astype(jnp.bfloat16)
        pl.semaphore_wait(barrier_sem, 2)

        s1_base = own + my_y * half
        sends1 = []
        for c in range(CH):
            rows = pl.ds(s1_base + c * r, r)
            rdma = pltpu.make_async_remote_copy(
                src_ref=out_ref.at[rows, :],
                dst_ref=out_ref.at[rows, :],
                send_sem=s1_send.at[c],
                recv_sem=s1_recv.at[c],
                device_id=xn,
                device_id_type=pl.DeviceIdType.MESH,
            )
            rdma.start()
            sends1.append(rdma)

        recv_base = opp + my_y * half
        sends2 = []
        for c in range(CH):
            rows = pl.ds(recv_base + c * r, r)
            inbound = pltpu.make_async_remote_copy(
                src_ref=out_ref.at[rows, :],
                dst_ref=out_ref.at[rows, :],
                send_sem=s1_send.at[c],
                recv_sem=s1_recv.at[c],
                device_id=xn,
                device_id_type=pl.DeviceIdType.MESH,
            )
            inbound.wait_recv()
            fwd = pltpu.make_async_remote_copy(
                src_ref=out_ref.at[rows, :],
                dst_ref=out_ref.at[rows, :],
                send_sem=s2_send.at[c],
                recv_sem=s2_recv.at[c],
                device_id=yn,
                device_id_type=pl.DeviceIdType.MESH,
            )
            fwd.start()
            sends2.append(fwd)

        fwd_base = opp + (1 - my_y) * half
        for c in range(CH):
            rows = pl.ds(fwd_base + c * r, r)
            inbound = pltpu.make_async_remote_copy(
                src_ref=out_ref.at[rows, :],
                dst_ref=out_ref.at[rows, :],
                send_sem=s2_send.at[c],
                recv_sem=s2_recv.at[c],
                device_id=yn,
                device_id_type=pl.DeviceIdType.MESH,
            )
            inbound.wait_recv()
        for rdma in sends1 + sends2:
            rdma.wait_send()

    return pl.pallas_call(
        body,
        out_shape=jax.ShapeDtypeStruct((2 * m, n), jnp.bfloat16),
        in_specs=[pl.BlockSpec(memory_space=pltpu.VMEM)],
        out_specs=pl.BlockSpec(memory_space=pltpu.VMEM),
        scratch_shapes=[
            pltpu.SemaphoreType.DMA((CH,)),
            pltpu.SemaphoreType.DMA((CH,)),
            pltpu.SemaphoreType.DMA((CH,)),
            pltpu.SemaphoreType.DMA((CH,)),
        ],
        compiler_params=pltpu.CompilerParams(collective_id=0),
    )(x)
